# baseline (speedup 1.0000x reference)
import functools

import jax
import jax.numpy as jnp
from jax import lax
from jax.experimental import pallas as pl
from jax.experimental.pallas import tpu as pltpu
from jax.experimental.pallas import tpu_sc as plsc


_NC = 2
_NS = 16
_NW = _NC * _NS


def _emb_kernel(n_rows, seq_len, d, chunk, idx_hbm, table_hbm, pos_hbm,
                out_hbm, idx_v, rows_v, pos2_v, sem):
    wid = lax.axis_index("s") * _NC + lax.axis_index("c")
    rows_per_w = n_rows // _NW
    n_chunks = rows_per_w // chunk
    base = wid * rows_per_w

    pltpu.sync_copy(pos_hbm, pos2_v.at[pl.ds(0, seq_len)])
    pltpu.sync_copy(pos_hbm, pos2_v.at[pl.ds(seq_len, seq_len)])

    nvec = d // 16

    @pl.loop(0, n_chunks)
    def _chunks(c):
        start = base + c * chunk
        pltpu.sync_copy(idx_hbm.at[pl.ds(start, chunk)], idx_v)
        pltpu.async_copy(table_hbm.at[idx_v], rows_v, sem).wait()
        off = lax.rem(start, seq_len)

        @pl.loop(0, chunk)
        def _rows(i):
            l = off + i
            for j in range(nvec):
                sl = pl.ds(j * 16, 16)
                rows_v[i, sl] += pos2_v[l, sl]

        pltpu.sync_copy(rows_v, out_hbm.at[pl.ds(start, chunk)])


def kernel(item_seq, item_table, token_mask, pos_table):
    del token_mask
    b, seq_len = item_seq.shape
    d = item_table.shape[1]
    n_rows = b * seq_len
    chunk = 128

    idx_flat = jnp.reshape(item_seq, (n_rows,)).astype(jnp.int32)

    mesh = plsc.VectorSubcoreMesh(core_axis_name="c", subcore_axis_name="s")
    fn = pl.kernel(
        functools.partial(_emb_kernel, n_rows, seq_len, d, chunk),
        out_type=jax.ShapeDtypeStruct((n_rows, d), jnp.float32),
        mesh=mesh,
        scratch_types=[
            pltpu.VMEM((chunk,), jnp.int32),
            pltpu.VMEM((chunk, d), jnp.float32),
            pltpu.VMEM((2 * seq_len, d), jnp.float32),
            pltpu.SemaphoreType.DMA,
        ],
        compiler_params=pltpu.CompilerParams(use_tc_tiling_on_sc=False),
    )
    out = fn(idx_flat, item_table, pos_table)
    return jnp.reshape(out, (b, seq_len, d))


# speedup vs baseline: 1.1656x; 1.1656x over previous
import functools

import jax
import jax.numpy as jnp
from jax import lax
from jax.experimental import pallas as pl
from jax.experimental.pallas import tpu as pltpu
from jax.experimental.pallas import tpu_sc as plsc


_NC = 2
_NS = 16
_NW = _NC * _NS
_CHUNK = 128
_K = 4


def _emb_kernel(n_rows, seq_len, d, idx_hbm, table_hbm, pos_hbm, out_hbm,
                idx_v, rows_v, pos2_v, gsems, wsems):
    wid = lax.axis_index("s") * _NC + lax.axis_index("c")
    rows_per_w = n_rows // _NW
    n_chunks = rows_per_w // _CHUNK
    base = wid * rows_per_w
    crow = wid * n_chunks

    pltpu.sync_copy(idx_hbm.at[pl.ds(crow, n_chunks)], idx_v)

    pltpu.sync_copy(pos_hbm, pos2_v.at[pl.ds(0, seq_len)])
    pltpu.sync_copy(pos_hbm, pos2_v.at[pl.ds(seq_len, seq_len)])

    nvec = d // 16

    def gather(c, b):
        pltpu.async_copy(table_hbm.at[idx_v.at[c]], rows_v.at[b], gsems[b])

    for b in range(_K):
        gather(b, b)

    @pl.loop(0, n_chunks)
    def _chunks(c):
        b = lax.rem(c, _K)
        start = base + c * _CHUNK
        off = lax.rem(start, seq_len)

        def per_buf(b):
            pltpu.make_async_copy(table_hbm.at[idx_v.at[c]],
                                  rows_v.at[b], gsems[b]).wait()

            @pl.loop(0, _CHUNK)
            def _rows(i):
                l = off + i
                for j in range(nvec):
                    sl = pl.ds(j * 16, 16)
                    rows_v[b, i, sl] += pos2_v[l, sl]

            pltpu.async_copy(rows_v.at[b], out_hbm.at[pl.ds(start, _CHUNK)],
                             wsems[b])
            nxt = c + _K

            @pl.when(nxt < n_chunks)
            def _():
                pltpu.make_async_copy(rows_v.at[b],
                                      out_hbm.at[pl.ds(start, _CHUNK)],
                                      wsems[b]).wait()
                gather(nxt, b)

        lax.switch(b, [functools.partial(per_buf, bb) for bb in range(_K)])

    for b in range(_K):
        pltpu.make_async_copy(rows_v.at[b], out_hbm.at[pl.ds(0, _CHUNK)],
                              wsems[b]).wait()


def kernel(item_seq, item_table, token_mask, pos_table):
    del token_mask
    b, seq_len = item_seq.shape
    d = item_table.shape[1]
    n_rows = b * seq_len

    idx2 = jnp.reshape(item_seq, (n_rows // _CHUNK, _CHUNK)).astype(jnp.int32)

    mesh = plsc.VectorSubcoreMesh(core_axis_name="c", subcore_axis_name="s")
    fn = pl.kernel(
        functools.partial(_emb_kernel, n_rows, seq_len, d),
        out_type=jax.ShapeDtypeStruct((n_rows, d), jnp.float32),
        mesh=mesh,
        scratch_types=[
            pltpu.VMEM((n_rows // _NW // _CHUNK, _CHUNK), jnp.int32),
            pltpu.VMEM((_K, _CHUNK, d), jnp.float32),
            pltpu.VMEM((2 * seq_len, d), jnp.float32),
            [pltpu.SemaphoreType.DMA] * _K,
            [pltpu.SemaphoreType.DMA] * _K,
        ],
        compiler_params=pltpu.CompilerParams(use_tc_tiling_on_sc=False),
    )
    out = fn(idx2, item_table, pos_table)
    return jnp.reshape(out, (b, seq_len, d))


# speedup vs baseline: 1.2146x; 1.0420x over previous
import functools

import jax
import jax.numpy as jnp
from jax import lax
from jax.experimental import pallas as pl
from jax.experimental.pallas import tpu as pltpu
from jax.experimental.pallas import tpu_sc as plsc


_NC = 2
_NS = 16
_NW = _NC * _NS
_IW = 128
_GROUP = 512
_NSUB = _GROUP // _IW


def _emb_kernel(n_rows, seq_len, d, idx_hbm, table_hbm, pos_hbm, out_hbm,
                idx_v, rows_v, pos2_v, gsems, wsems):
    wid = lax.axis_index("s") * _NC + lax.axis_index("c")
    rows_per_w = n_rows // _NW
    n_groups = rows_per_w // _GROUP
    base = wid * rows_per_w
    irow = wid * (rows_per_w // _IW)

    pltpu.sync_copy(idx_hbm.at[pl.ds(irow, rows_per_w // _IW)], idx_v)

    pltpu.sync_copy(pos_hbm, pos2_v.at[pl.ds(0, seq_len)])
    pltpu.sync_copy(pos_hbm, pos2_v.at[pl.ds(seq_len, seq_len)])

    nvec = d // 16

    def gathers(g, b):
        for q in range(_NSUB):
            pltpu.async_copy(table_hbm.at[idx_v.at[g * _NSUB + q]],
                             rows_v.at[b, pl.ds(q * _IW, _IW)], gsems[b])

    gathers(0, 0)

    @pl.loop(0, n_groups)
    def _groups(g):
        start = base + g * _GROUP

        def per_buf(b):
            nb = 1 - b
            for q in range(_NSUB):
                pltpu.make_async_copy(
                    table_hbm.at[idx_v.at[g * _NSUB + q]],
                    rows_v.at[b, pl.ds(q * _IW, _IW)], gsems[b]).wait()

            @pl.when(g > 0)
            def _():
                pltpu.make_async_copy(
                    rows_v.at[nb], out_hbm.at[pl.ds(0, _GROUP)],
                    wsems[nb]).wait()

            @pl.when(g + 1 < n_groups)
            def _():
                gathers(g + 1, nb)

            for q in range(_NSUB):
                off = lax.rem(start + q * _IW, seq_len)
                qb = q * _IW

                @pl.loop(0, _IW, unroll=4)
                def _rows(i):
                    l = off + i
                    r = qb + i
                    for j in range(nvec):
                        sl = pl.ds(j * 16, 16)
                        rows_v[b, r, sl] += pos2_v[l, sl]

            pltpu.async_copy(rows_v.at[b], out_hbm.at[pl.ds(start, _GROUP)],
                             wsems[b])

        lax.switch(lax.rem(g, 2), [functools.partial(per_buf, 0),
                                   functools.partial(per_buf, 1)])

    last = (n_groups - 1) % 2
    pltpu.make_async_copy(rows_v.at[last], out_hbm.at[pl.ds(0, _GROUP)],
                          wsems[last]).wait()


def kernel(item_seq, item_table, token_mask, pos_table):
    del token_mask
    b, seq_len = item_seq.shape
    d = item_table.shape[1]
    n_rows = b * seq_len

    idx2 = jnp.reshape(item_seq, (n_rows // _IW, _IW)).astype(jnp.int32)

    mesh = plsc.VectorSubcoreMesh(core_axis_name="c", subcore_axis_name="s")
    fn = pl.kernel(
        functools.partial(_emb_kernel, n_rows, seq_len, d),
        out_type=jax.ShapeDtypeStruct((n_rows, d), jnp.float32),
        mesh=mesh,
        scratch_types=[
            pltpu.VMEM((n_rows // _NW // _IW, _IW), jnp.int32),
            pltpu.VMEM((2, _GROUP, d), jnp.float32),
            pltpu.VMEM((2 * seq_len, d), jnp.float32),
            [pltpu.SemaphoreType.DMA] * 2,
            [pltpu.SemaphoreType.DMA] * 2,
        ],
        compiler_params=pltpu.CompilerParams(use_tc_tiling_on_sc=False),
    )
    out = fn(idx2, item_table, pos_table)
    return jnp.reshape(out, (b, seq_len, d))


# speedup vs baseline: 1.3714x; 1.1291x over previous
import functools

import jax
import jax.numpy as jnp
from jax import lax
from jax.experimental import pallas as pl
from jax.experimental.pallas import tpu as pltpu
from jax.experimental.pallas import tpu_sc as plsc


_NC = 2
_NS = 16
_NW = _NC * _NS
_SPLIT = 104
_SEQ_PER_G = 2


def _emb_kernel(n_b, seq_len, d, idx_hbm, table_hbm, pos_hbm, out_hbm,
                idx_v, rows_v, pos_v, gsems, wsems):
    wid = lax.axis_index("s") * _NC + lax.axis_index("c")
    b_per_w = n_b // _NW
    n_groups = b_per_w // _SEQ_PER_G
    grows = _SEQ_PER_G * seq_len
    b0 = wid * b_per_w

    pltpu.sync_copy(idx_hbm.at[pl.ds(b0, b_per_w)], idx_v)
    pltpu.sync_copy(pos_hbm, pos_v)

    nvec = d // 16
    halves = ((0, _SPLIT), (_SPLIT, seq_len - _SPLIT))

    def gathers(g, bf):
        for s in range(_SEQ_PER_G):
            r = g * _SEQ_PER_G + s
            for off, ln in halves:
                pltpu.async_copy(
                    table_hbm.at[idx_v.at[r, pl.ds(off, ln)]],
                    rows_v.at[bf, pl.ds(s * seq_len + off, ln)], gsems[bf])

    gathers(0, 0)

    @pl.loop(0, n_groups)
    def _groups(g):
        start = (b0 + g * _SEQ_PER_G) * seq_len

        def per_buf(bf):
            nb = 1 - bf
            for s in range(_SEQ_PER_G):
                r = g * _SEQ_PER_G + s
                for off, ln in halves:
                    pltpu.make_async_copy(
                        table_hbm.at[idx_v.at[r, pl.ds(off, ln)]],
                        rows_v.at[bf, pl.ds(s * seq_len + off, ln)],
                        gsems[bf]).wait()

            @pl.when(g > 0)
            def _():
                pltpu.make_async_copy(
                    rows_v.at[nb], out_hbm.at[pl.ds(0, grows)],
                    wsems[nb]).wait()

            @pl.when(g + 1 < n_groups)
            def _():
                gathers(g + 1, nb)

            for s in range(_SEQ_PER_G):
                sb = s * seq_len

                @pl.loop(0, seq_len, unroll=4)
                def _rows(i):
                    for j in range(nvec):
                        sl = pl.ds(j * 16, 16)
                        rows_v[bf, sb + i, sl] += pos_v[i, sl]

            pltpu.async_copy(rows_v.at[bf], out_hbm.at[pl.ds(start, grows)],
                             wsems[bf])

        lax.switch(lax.rem(g, 2), [functools.partial(per_buf, 0),
                                   functools.partial(per_buf, 1)])

    last = (n_groups - 1) % 2
    pltpu.make_async_copy(rows_v.at[last], out_hbm.at[pl.ds(0, grows)],
                          wsems[last]).wait()


def kernel(item_seq, item_table, token_mask, pos_table):
    del token_mask
    n_b, seq_len = item_seq.shape
    d = item_table.shape[1]

    mesh = plsc.VectorSubcoreMesh(core_axis_name="c", subcore_axis_name="s")
    fn = pl.kernel(
        functools.partial(_emb_kernel, n_b, seq_len, d),
        out_type=jax.ShapeDtypeStruct((n_b * seq_len, d), jnp.float32),
        mesh=mesh,
        scratch_types=[
            pltpu.VMEM((n_b // _NW, seq_len), jnp.int32),
            pltpu.VMEM((2, _SEQ_PER_G * seq_len, d), jnp.float32),
            pltpu.VMEM((seq_len, d), jnp.float32),
            [pltpu.SemaphoreType.DMA] * 2,
            [pltpu.SemaphoreType.DMA] * 2,
        ],
        compiler_params=pltpu.CompilerParams(use_tc_tiling_on_sc=False),
    )
    out = fn(item_seq.astype(jnp.int32), item_table, pos_table)
    return jnp.reshape(out, (n_b, seq_len, d))
